# baseline (speedup 1.0000x reference)
import math

import jax
import jax.numpy as jnp
from jax.experimental import pallas as pl
from jax.experimental.pallas import tpu as pltpu


def _round_up(a, b):
    return (a + b - 1) // b * b


def _cdiv(a, b):
    return -(-a // b)


def _neg_min(dtype):
    dtype = jnp.dtype(dtype)
    if jnp.issubdtype(dtype, jnp.floating):
        return float("-inf")
    if jnp.issubdtype(dtype, jnp.integer):
        return int(jnp.iinfo(dtype).min)
    raise ValueError(f"unsupported dtype for max pooling: {dtype}")


def _fold(arr, num_groups, last_valid, min_val):
    m = None
    for g in range(num_groups):
        blk = arr[:, g * 128:(g + 1) * 128]
        if g == num_groups - 1 and last_valid < 128:
            lane = jax.lax.broadcasted_iota(jnp.int32, blk.shape, 1)
            blk = jnp.where(lane < last_valid, blk,
                            jnp.full_like(blk, min_val))
        m = blk if m is None else jnp.maximum(m, blk)
    return m


def _make_ring_body(rows_per_prog, chunk, num_groups, nbuf, min_val):
    num_chunks = rows_per_prog // chunk

    def body(x_hbm, o_ref, buf, sem):
        base = pl.program_id(0) * rows_per_prog

        def start(j):
            pltpu.make_async_copy(
                x_hbm.at[pl.ds(base + j * chunk, chunk), :],
                buf.at[j % nbuf],
                sem.at[j % nbuf],
            ).start()

        for j in range(min(nbuf, num_chunks)):
            start(j)
        for j in range(num_chunks):
            pltpu.make_async_copy(
                x_hbm.at[pl.ds(base + j * chunk, chunk), :],
                buf.at[j % nbuf],
                sem.at[j % nbuf],
            ).wait()
            m = _fold(buf[j % nbuf], num_groups, 128, min_val)
            o_ref[pl.ds(j * chunk, chunk), :] = jnp.max(
                m, axis=-1, keepdims=True
            ).astype(o_ref.dtype)
            if j + nbuf < num_chunks:
                start(j + nbuf)

    return body


def _make_auto_body(num_groups, last_valid, min_val):
    def body(x_ref, o_ref):
        m = _fold(x_ref[...], num_groups, last_valid, min_val)
        o_ref[...] = jnp.max(m, axis=-1, keepdims=True).astype(o_ref.dtype)

    return body


def _global_max_last_axis(x):
    *lead, L = x.shape
    R = math.prod(lead) if lead else 1
    out_shape = tuple(lead)

    itemsize = jnp.dtype(x.dtype).itemsize
    sub = {4: 8, 2: 16, 1: 32}.get(itemsize, 8)
    Lp = _round_up(L, 128)
    num_groups = Lp // 128
    last_valid = L - (num_groups - 1) * 128

    xf = x.reshape(R, L)
    min_val = _neg_min(x.dtype)

    chunk = 1024
    nbuf = 4
    if (L == Lp and R % (2 * chunk) == 0
            and chunk * L * itemsize * nbuf <= 24 * 1024 * 1024):
        rows_per_prog = R // 2
        out = pl.pallas_call(
            _make_ring_body(rows_per_prog, chunk, num_groups, nbuf, min_val),
            out_shape=jax.ShapeDtypeStruct((R, 1), x.dtype),
            grid=(2,),
            in_specs=[pl.BlockSpec(memory_space=pl.ANY)],
            out_specs=pl.BlockSpec((rows_per_prog, 1), lambda i: (i, 0)),
            scratch_shapes=[
                pltpu.VMEM((nbuf, chunk, L), x.dtype),
                pltpu.SemaphoreType.DMA((nbuf,)),
            ],
            compiler_params=pltpu.CompilerParams(
                dimension_semantics=("parallel",),
                vmem_limit_bytes=48 * 1024 * 1024,
            ),
        )(xf)
        return out[:, 0].reshape(out_shape)

    budget = 8 * 1024 * 1024
    TR = max(sub, min(_round_up(R, sub), 2048,
                      (budget // (Lp * itemsize)) // sub * sub))
    if _cdiv(R, TR) < 2 and R > sub:
        TR = _round_up(_cdiv(R, 2), sub)
    num_r = _cdiv(R, TR)

    out = pl.pallas_call(
        _make_auto_body(num_groups, last_valid, min_val),
        out_shape=jax.ShapeDtypeStruct((R, 1), x.dtype),
        grid=(num_r,),
        in_specs=[pl.BlockSpec((TR, Lp), lambda i: (i, 0))],
        out_specs=pl.BlockSpec((TR, 1), lambda i: (i, 0)),
        compiler_params=pltpu.CompilerParams(
            dimension_semantics=("parallel",),
            vmem_limit_bytes=48 * 1024 * 1024,
        ),
    )(xf)
    return out[:, 0].reshape(out_shape)


def kernel(x):
    return _global_max_last_axis(x)


# speedup vs baseline: 1.3497x; 1.3497x over previous
import math

import jax
import jax.numpy as jnp
from jax.experimental import pallas as pl
from jax.experimental.pallas import tpu as pltpu


def _round_up(a, b):
    return (a + b - 1) // b * b


def _cdiv(a, b):
    return -(-a // b)


def _neg_min(dtype):
    dtype = jnp.dtype(dtype)
    if jnp.issubdtype(dtype, jnp.floating):
        return float("-inf")
    if jnp.issubdtype(dtype, jnp.integer):
        return int(jnp.iinfo(dtype).min)
    raise ValueError(f"unsupported dtype for max pooling: {dtype}")


def _fold(arr, num_groups, last_valid, min_val):
    m = None
    for g in range(num_groups):
        blk = arr[:, g * 128:(g + 1) * 128]
        if g == num_groups - 1 and last_valid < 128:
            lane = jax.lax.broadcasted_iota(jnp.int32, blk.shape, 1)
            blk = jnp.where(lane < last_valid, blk,
                            jnp.full_like(blk, min_val))
        m = blk if m is None else jnp.maximum(m, blk)
    return m


def _make_wide_body(TR, C, num_groups, last_valid, min_val):
    def body(x_ref, o_ref):
        m = _fold(x_ref[...], num_groups, last_valid, min_val)
        for a in range(TR // C):
            s = jnp.transpose(m[a * C:(a + 1) * C, :])
            o_ref[pl.ds(a, 1), :] = jnp.max(
                s, axis=0, keepdims=True
            ).astype(o_ref.dtype)

    return body


def _make_col_body(num_groups, last_valid, min_val):
    def body(x_ref, o_ref):
        m = _fold(x_ref[...], num_groups, last_valid, min_val)
        o_ref[...] = jnp.max(m, axis=-1, keepdims=True).astype(o_ref.dtype)

    return body


def _global_max_last_axis(x):
    *lead, L = x.shape
    R = math.prod(lead) if lead else 1
    out_shape = tuple(lead)

    itemsize = jnp.dtype(x.dtype).itemsize
    sub = {4: 8, 2: 16, 1: 32}.get(itemsize, 8)
    Lp = _round_up(L, 128)
    num_groups = Lp // 128
    last_valid = L - (num_groups - 1) * 128

    budget = 8 * 1024 * 1024
    TR = max(sub, min(_round_up(R, sub), 2048,
                      (budget // (Lp * itemsize)) // sub * sub))
    if _cdiv(R, TR) < 2 and R > sub:
        TR = _round_up(_cdiv(R, 2), sub)
    num_r = _cdiv(R, TR)

    xf = x.reshape(R, L)
    min_val = _neg_min(x.dtype)
    common = dict(
        grid=(num_r,),
        in_specs=[pl.BlockSpec((TR, Lp), lambda i: (i, 0))],
        compiler_params=pltpu.CompilerParams(
            dimension_semantics=("parallel",),
            vmem_limit_bytes=48 * 1024 * 1024,
        ),
    )

    C = x.shape[-2] if x.ndim >= 3 else 0
    if C and C % 128 == 0 and TR % C == 0 and R % TR == 0:
        out = pl.pallas_call(
            _make_wide_body(TR, C, num_groups, last_valid, min_val),
            out_shape=jax.ShapeDtypeStruct((R // C, C), x.dtype),
            out_specs=pl.BlockSpec((TR // C, C), lambda i: (i, 0)),
            **common,
        )(xf)
        return out.reshape(out_shape)

    out = pl.pallas_call(
        _make_col_body(num_groups, last_valid, min_val),
        out_shape=jax.ShapeDtypeStruct((R, 1), x.dtype),
        out_specs=pl.BlockSpec((TR, 1), lambda i: (i, 0)),
        **common,
    )(xf)
    return out[:, 0].reshape(out_shape)


def kernel(x):
    return _global_max_last_axis(x)


# speedup vs baseline: 1.3828x; 1.0245x over previous
import math

import jax
import jax.numpy as jnp
from jax.experimental import pallas as pl
from jax.experimental.pallas import tpu as pltpu


def _round_up(a, b):
    return (a + b - 1) // b * b


def _cdiv(a, b):
    return -(-a // b)


def _neg_min(dtype):
    dtype = jnp.dtype(dtype)
    if jnp.issubdtype(dtype, jnp.floating):
        return float("-inf")
    if jnp.issubdtype(dtype, jnp.integer):
        return int(jnp.iinfo(dtype).min)
    raise ValueError(f"unsupported dtype for max pooling: {dtype}")


def _fold(arr, num_groups, last_valid, min_val):
    m = None
    for g in range(num_groups):
        blk = arr[:, g * 128:(g + 1) * 128]
        if g == num_groups - 1 and last_valid < 128:
            lane = jax.lax.broadcasted_iota(jnp.int32, blk.shape, 1)
            blk = jnp.where(lane < last_valid, blk,
                            jnp.full_like(blk, min_val))
        m = blk if m is None else jnp.maximum(m, blk)
    return m


def _make_wide_body(TR, C, num_groups, last_valid, min_val):
    def body(x_ref, o_ref):
        m = _fold(x_ref[...], num_groups, last_valid, min_val)
        for a in range(TR // C):
            s = jnp.transpose(m[a * C:(a + 1) * C, :])
            o_ref[pl.ds(a, 1), :] = jnp.max(
                s, axis=0, keepdims=True
            ).astype(o_ref.dtype)

    return body


def _make_col_body(num_groups, last_valid, min_val):
    def body(x_ref, o_ref):
        m = _fold(x_ref[...], num_groups, last_valid, min_val)
        o_ref[...] = jnp.max(m, axis=-1, keepdims=True).astype(o_ref.dtype)

    return body


def _global_max_last_axis(x):
    *lead, L = x.shape
    R = math.prod(lead) if lead else 1
    out_shape = tuple(lead)

    itemsize = jnp.dtype(x.dtype).itemsize
    sub = {4: 8, 2: 16, 1: 32}.get(itemsize, 8)
    Lp = _round_up(L, 128)
    num_groups = Lp // 128
    last_valid = L - (num_groups - 1) * 128

    budget = 16 * 1024 * 1024
    TR = max(sub, min(_round_up(R, sub), 4096,
                      (budget // (Lp * itemsize)) // sub * sub))
    if _cdiv(R, TR) < 2 and R > sub:
        TR = _round_up(_cdiv(R, 2), sub)
    num_r = _cdiv(R, TR)

    xf = x.reshape(R, L)
    min_val = _neg_min(x.dtype)
    common = dict(
        grid=(num_r,),
        in_specs=[pl.BlockSpec((TR, Lp), lambda i: (i, 0))],
        compiler_params=pltpu.CompilerParams(
            dimension_semantics=("parallel",),
            vmem_limit_bytes=48 * 1024 * 1024,
        ),
    )

    C = x.shape[-2] if x.ndim >= 3 else 0
    if C and C % 128 == 0 and TR % C == 0 and R % TR == 0:
        out = pl.pallas_call(
            _make_wide_body(TR, C, num_groups, last_valid, min_val),
            out_shape=jax.ShapeDtypeStruct((R // C, C), x.dtype),
            out_specs=pl.BlockSpec((TR // C, C), lambda i: (i, 0)),
            **common,
        )(xf)
        return out.reshape(out_shape)

    out = pl.pallas_call(
        _make_col_body(num_groups, last_valid, min_val),
        out_shape=jax.ShapeDtypeStruct((R, 1), x.dtype),
        out_specs=pl.BlockSpec((TR, 1), lambda i: (i, 0)),
        **common,
    )(xf)
    return out[:, 0].reshape(out_shape)


def kernel(x):
    return _global_max_last_axis(x)
